# baseline (speedup 1.0000x reference)
import functools

import jax
import jax.numpy as jnp
from jax import lax
from jax.experimental import pallas as pl
from jax.experimental.pallas import tpu as pltpu
from jax.experimental.pallas import tpu_sc as plsc

_TAU = 0.9

_NC = 2
_NS = 16
_L = 16
_NW = _NC * _NS
_ROWS = 128
_N = 8192
_RPW = _ROWS // _NW
_NV = _N // _L
_B1 = 1024
_MACB = _B1 // _L
_W = 11.5
_CAND = 32


def _make_sc_kernel():
    mesh = plsc.VectorSubcoreMesh(
        core_axis_name="c", subcore_axis_name="s",
        num_cores=_NC, num_subcores=_NS)

    @functools.partial(
        pl.kernel,
        out_type=jax.ShapeDtypeStruct((_ROWS, _N), jnp.float32),
        mesh=mesh,
        compiler_params=pltpu.CompilerParams(needs_layout_passes=False),
        scratch_types=[
            pltpu.VMEM((_N,), jnp.float32),
            pltpu.VMEM((_N,), jnp.float32),
            pltpu.VMEM((_B1 * _L,), jnp.float32),
            pltpu.VMEM((_L, _L), jnp.float32),
            pltpu.VMEM((_CAND * _L,), jnp.float32),
            pltpu.VMEM((_CAND * _L,), jnp.int32),
            pltpu.VMEM((_CAND * _L,), jnp.float32),
            pltpu.VMEM((_CAND * _L,), jnp.int32),
        ],
    )
    def sc_kernel(z_hbm, out_hbm, zrow, orow, hist, coarse,
                  cand_v, cand_i, cand_e, cand_k):
        imin = jnp.int32(-2147483648)
        lane = lax.broadcasted_iota(jnp.int32, (_L,), 0)
        lanef = lane.astype(jnp.float32)
        zeros = jnp.zeros((_L,), jnp.float32)
        izeros = jnp.zeros((_L,), jnp.int32)
        wid = lax.axis_index("s") * _NC + lax.axis_index("c")

        def zinit(i, c):
            hist[pl.ds(i * _L, _L)] = zeros
            return c
        lax.fori_loop(0, _B1, zinit, 0)

        def do_row(r, carry):
            row = wid * _RPW + r
            pltpu.sync_copy(z_hbm.at[row], zrow)

            def pmax(i, acc):
                return jnp.maximum(acc, zrow[pl.ds(i * _L, _L)])
            vm = lax.fori_loop(0, _NV, pmax,
                               jnp.full((_L,), -jnp.inf, jnp.float32))
            m = jnp.max(vm)
            lo = m - jnp.float32(_W)
            scale = jnp.float32(_B1 / _W)

            def czero(l, c):
                coarse[l] = zeros
                return c
            lax.fori_loop(0, _L, czero, 0)

            def p2(i, acc):
                zv = zrow[pl.ds(i * _L, _L)]
                ev = jnp.exp(zv - m)
                b = jnp.clip(((zv - lo) * scale).astype(jnp.int32),
                             0, _B1 - 1)
                plsc.addupdate_scatter(hist, [b * _L + lane], ev)
                plsc.addupdate_scatter(coarse, [lane, b >> 6], ev)
                return acc + ev
            sv = lax.fori_loop(0, _NV, p2, zeros)
            s = jnp.sum(sv)
            t = jnp.float32(_TAU) * s

            def ct(l, acc):
                return acc + coarse[l]
            tot = lax.fori_loop(0, _L, ct, zeros)
            pref = plsc.cumsum(tot)
            s_c = jnp.max(pref)
            suf = s_c - pref + tot
            cond = suf >= t
            mstar = jnp.max(jnp.where(cond, lane, -1))
            suf_at = jnp.sum(jnp.where(lane == mstar, suf, 0.0))
            tot_at = jnp.sum(jnp.where(lane == mstar, tot, 0.0))
            g_mac = suf_at - tot_at

            def chunk_tot(c):
                base = (mstar * _MACB + c * _L) * _L + lane * _L
                def gt(i, acc):
                    return acc + plsc.load_gather(hist, [base + i])
                return lax.fori_loop(0, _L, gt, zeros)

            def cscan(i, carry):
                g, bstar, g_ab, found = carry
                c = 3 - i
                t16 = chunk_tot(c)
                pref16 = plsc.cumsum(t16)
                ctot = jnp.max(pref16)
                suf16 = ctot - pref16 + t16
                cnd = (g + suf16) >= t
                bl = jnp.max(jnp.where(cnd, lane, -1))
                hit = jnp.logical_and(jnp.logical_not(found), bl >= 0)
                pref_at = jnp.sum(jnp.where(lane == bl, pref16, 0.0))
                bstar = jnp.where(hit, mstar * _MACB + c * _L + bl, bstar)
                g_ab = jnp.where(hit, g + (ctot - pref_at), g_ab)
                found = jnp.logical_or(found, hit)
                g = jnp.where(found, g, g + ctot)
                return g, bstar, g_ab, found
            _, bstar, g_ab, _ = lax.fori_loop(
                0, 4, cscan,
                (g_mac, jnp.int32(0), jnp.float32(0.0), False))

            def p3(i, cnt):
                zv = zrow[pl.ds(i * _L, _L)]
                b = jnp.clip(((zv - lo) * scale).astype(jnp.int32),
                             0, _B1 - 1)
                msk = b == bstar
                slot = jnp.minimum(cnt, _CAND - 1) * _L + lane
                plsc.store_scatter(cand_v, [slot], zv, mask=msk)
                plsc.store_scatter(cand_i, [slot], i * _L + lane, mask=msk)
                return cnt + msk.astype(jnp.int32)
            cnt = lax.fori_loop(0, _NV, p3, izeros)
            maxcnt = jnp.minimum(jnp.max(cnt), _CAND)

            def pk(si, c):
                vv = cand_v[pl.ds(si * _L, _L)]
                cand_e[pl.ds(si * _L, _L)] = jnp.exp(vv - m)
                bb = plsc.bitcast(vv, jnp.int32)
                uk = jnp.where(bb >= 0, bb | imin, ~bb)
                cand_k[pl.ds(si * _L, _L)] = uk ^ imin
                return c
            lax.fori_loop(0, maxcnt, pk, 0)

            def ctot_f(si, acc):
                valid = cnt > si
                ee = cand_e[pl.ds(si * _L, _L)]
                return acc + jnp.where(valid, ee, 0.0)
            c_tot = jnp.sum(lax.fori_loop(0, maxcnt, ctot_f, zeros))
            target = jnp.minimum(t - g_ab, c_tot)

            def hsum(ks):
                def inner(si, acc):
                    valid = cnt > si
                    sk = cand_k[pl.ds(si * _L, _L)]
                    ee = cand_e[pl.ds(si * _L, _L)]
                    sel = jnp.logical_and(valid, sk >= ks)
                    return acc + jnp.where(sel, ee, 0.0)
                return jnp.sum(lax.fori_loop(0, maxcnt, inner, zeros))

            def bstep(bi, k):
                kc = k | (jnp.int32(1) << (31 - bi))
                h = hsum(kc ^ imin)
                return jnp.where(h >= target, kc, k)
            kk = lax.fori_loop(0, 32, bstep, jnp.int32(0))

            ks = kk ^ imin
            fbits = jnp.where(kk < 0, kk & (~imin), ~kk)
            fb_v = jnp.full((_L,), fbits, jnp.int32)
            vstar_v = plsc.bitcast(fb_v, jnp.float32)
            vstar = jnp.max(vstar_v)
            estar_v = jnp.exp(vstar_v - m)

            def ggt_f(si, acc):
                valid = cnt > si
                sk = cand_k[pl.ds(si * _L, _L)]
                ee = cand_e[pl.ds(si * _L, _L)]
                sel = jnp.logical_and(valid, sk > ks)
                return acc + jnp.where(sel, ee, 0.0)
            g_gt = g_ab + jnp.sum(lax.fori_loop(0, maxcnt, ggt_f, zeros))
            x = jnp.max((t - g_gt) / estar_v)
            jt = x.astype(jnp.int32).astype(jnp.float32)
            j = jt + (jt < x).astype(jnp.float32)

            def teq(ic):
                def inner(si, acc):
                    valid = cnt > si
                    sk = cand_k[pl.ds(si * _L, _L)]
                    ii = cand_i[pl.ds(si * _L, _L)]
                    sel = jnp.logical_and(
                        valid, jnp.logical_and(sk == ks, ii <= ic))
                    return acc + jnp.where(sel, 1.0, 0.0)
                return jnp.sum(lax.fori_loop(0, maxcnt, inner, zeros))

            def tstep(bi, iacc):
                ic = iacc | (jnp.int32(1) << (12 - bi))
                return jnp.where(teq(ic) < j, ic, iacc)
            iacc = lax.fori_loop(0, 13, tstep, jnp.int32(0))
            bound = jnp.where(teq(iacc) < j, iacc + 1, iacc)

            def pf(i, c):
                zv = zrow[pl.ds(i * _L, _L)]
                b = jnp.clip(((zv - lo) * scale).astype(jnp.int32),
                             0, _B1 - 1)
                plsc.store_scatter(hist, [b * _L + lane], zeros)
                idxv = i * _L + lane
                keep = jnp.logical_or(
                    zv > vstar,
                    jnp.logical_and(zv == vstar, idxv <= bound))
                orow[pl.ds(i * _L, _L)] = jnp.where(
                    keep, jnp.maximum(zv, 0.0), 0.0)
                return c
            lax.fori_loop(0, _NV, pf, 0)

            pltpu.sync_copy(orow, out_hbm.at[row])
            return carry

        lax.fori_loop(0, _RPW, do_row, 0)

    return sc_kernel


_sc_cache = []


@jax.jit
def kernel(z):
    if not _sc_cache:
        _sc_cache.append(_make_sc_kernel())
    return _sc_cache[0](z)


# speedup vs baseline: 1.0751x; 1.0751x over previous
import functools

import jax
import jax.numpy as jnp
from jax import lax
from jax.experimental import pallas as pl
from jax.experimental.pallas import tpu as pltpu
from jax.experimental.pallas import tpu_sc as plsc

_TAU = 0.9

_NC = 2
_NS = 16
_L = 16
_NW = _NC * _NS
_ROWS = 128
_N = 8192
_RPW = _ROWS // _NW
_NV = _N // _L
_B1 = 1024
_MACB = _B1 // _L
_W = 11.5
_CAND = 32


def _make_sc_kernel():
    mesh = plsc.VectorSubcoreMesh(
        core_axis_name="c", subcore_axis_name="s",
        num_cores=_NC, num_subcores=_NS)

    @functools.partial(
        pl.kernel,
        out_type=jax.ShapeDtypeStruct((_ROWS, _N), jnp.float32),
        mesh=mesh,
        compiler_params=pltpu.CompilerParams(needs_layout_passes=False),
        scratch_types=[
            pltpu.VMEM((_N,), jnp.float32),
            pltpu.VMEM((_N,), jnp.float32),
            pltpu.VMEM((_B1 * _L,), jnp.float32),
            pltpu.VMEM((_L, _L), jnp.float32),
            pltpu.VMEM((_CAND * _L,), jnp.float32),
            pltpu.VMEM((_CAND * _L,), jnp.int32),
            pltpu.VMEM((_CAND * _L,), jnp.float32),
            pltpu.VMEM((_CAND * _L,), jnp.int32),
        ],
    )
    def sc_kernel(z_hbm, out_hbm, zrow, orow, hist, coarse,
                  cand_v, cand_i, cand_e, cand_k):
        imin = jnp.int32(-2147483648)
        lane = lax.broadcasted_iota(jnp.int32, (_L,), 0)
        lanef = lane.astype(jnp.float32)
        zeros = jnp.zeros((_L,), jnp.float32)
        izeros = jnp.zeros((_L,), jnp.int32)
        wid = lax.axis_index("s") * _NC + lax.axis_index("c")

        def zinit(i, c):
            for u in range(8):
                hist[pl.ds(i * (8 * _L) + u * _L, _L)] = zeros
            return c
        lax.fori_loop(0, _B1 // 8, zinit, 0)

        def do_row(r, carry):
            row = wid * _RPW + r
            pltpu.sync_copy(z_hbm.at[row], zrow)

            def pmax(i, acc):
                b0 = i * (4 * _L)
                a01 = jnp.maximum(zrow[pl.ds(b0, _L)],
                                  zrow[pl.ds(b0 + _L, _L)])
                a23 = jnp.maximum(zrow[pl.ds(b0 + 2 * _L, _L)],
                                  zrow[pl.ds(b0 + 3 * _L, _L)])
                return jnp.maximum(acc, jnp.maximum(a01, a23))
            vm = lax.fori_loop(0, _NV // 4, pmax,
                               jnp.full((_L,), -jnp.inf, jnp.float32))
            m = jnp.max(vm)
            lo = m - jnp.float32(_W)
            scale = jnp.float32(_B1 / _W)

            def czero(l, c):
                coarse[l] = zeros
                return c
            lax.fori_loop(0, _L, czero, 0)

            def p2(i, acc):
                a = acc
                for u in range(4):
                    zv = zrow[pl.ds(i * (4 * _L) + u * _L, _L)]
                    ev = jnp.exp(zv - m)
                    b = jnp.clip(((zv - lo) * scale).astype(jnp.int32),
                                 0, _B1 - 1)
                    plsc.addupdate_scatter(hist, [b * _L + lane], ev)
                    plsc.addupdate_scatter(coarse, [lane, b >> 6], ev)
                    a = a + ev
                return a
            sv = lax.fori_loop(0, _NV // 4, p2, zeros)
            s = jnp.sum(sv)
            t = jnp.float32(_TAU) * s

            def ct(l, acc):
                return acc + coarse[l]
            tot = lax.fori_loop(0, _L, ct, zeros)
            pref = plsc.cumsum(tot)
            s_c = jnp.max(pref)
            suf = s_c - pref + tot
            cond = suf >= t
            mstar = jnp.max(jnp.where(cond, lane, -1))
            suf_at = jnp.sum(jnp.where(lane == mstar, suf, 0.0))
            tot_at = jnp.sum(jnp.where(lane == mstar, tot, 0.0))
            g_mac = suf_at - tot_at

            def chunk_tot(c):
                base = (mstar * _MACB + c * _L) * _L + lane * _L
                def gt(i, acc):
                    return acc + plsc.load_gather(hist, [base + i])
                return lax.fori_loop(0, _L, gt, zeros)

            def cscan(i, carry):
                g, bstar, g_ab, found = carry
                c = 3 - i
                t16 = chunk_tot(c)
                pref16 = plsc.cumsum(t16)
                ctot = jnp.max(pref16)
                suf16 = ctot - pref16 + t16
                cnd = (g + suf16) >= t
                bl = jnp.max(jnp.where(cnd, lane, -1))
                hit = jnp.logical_and(jnp.logical_not(found), bl >= 0)
                pref_at = jnp.sum(jnp.where(lane == bl, pref16, 0.0))
                bstar = jnp.where(hit, mstar * _MACB + c * _L + bl, bstar)
                g_ab = jnp.where(hit, g + (ctot - pref_at), g_ab)
                found = jnp.logical_or(found, hit)
                g = jnp.where(found, g, g + ctot)
                return g, bstar, g_ab, found
            _, bstar, g_ab, _ = lax.fori_loop(
                0, 4, cscan,
                (g_mac, jnp.int32(0), jnp.float32(0.0), False))

            def p3(i, cnt):
                c = cnt
                for u in range(4):
                    zv = zrow[pl.ds(i * (4 * _L) + u * _L, _L)]
                    b = jnp.clip(((zv - lo) * scale).astype(jnp.int32),
                                 0, _B1 - 1)
                    msk = b == bstar
                    slot = jnp.minimum(c, _CAND - 1) * _L + lane
                    plsc.store_scatter(cand_v, [slot], zv, mask=msk)
                    plsc.store_scatter(cand_i, [slot],
                                       (i * 4 + u) * _L + lane, mask=msk)
                    c = c + msk.astype(jnp.int32)
                return c
            cnt = lax.fori_loop(0, _NV // 4, p3, izeros)
            maxcnt = jnp.minimum(jnp.max(cnt), _CAND)

            def pk(si, c):
                vv = cand_v[pl.ds(si * _L, _L)]
                cand_e[pl.ds(si * _L, _L)] = jnp.exp(vv - m)
                bb = plsc.bitcast(vv, jnp.int32)
                uk = jnp.where(bb >= 0, bb | imin, ~bb)
                cand_k[pl.ds(si * _L, _L)] = uk ^ imin
                return c
            lax.fori_loop(0, maxcnt, pk, 0)

            def ctot_f(si, acc):
                valid = cnt > si
                ee = cand_e[pl.ds(si * _L, _L)]
                return acc + jnp.where(valid, ee, 0.0)
            c_tot = jnp.sum(lax.fori_loop(0, maxcnt, ctot_f, zeros))
            target = jnp.minimum(t - g_ab, c_tot)

            def hsum(ks):
                def inner(si, acc):
                    valid = cnt > si
                    sk = cand_k[pl.ds(si * _L, _L)]
                    ee = cand_e[pl.ds(si * _L, _L)]
                    sel = jnp.logical_and(valid, sk >= ks)
                    return acc + jnp.where(sel, ee, 0.0)
                return jnp.sum(lax.fori_loop(0, maxcnt, inner, zeros))

            def bstep(bi, k):
                kc = k | (jnp.int32(1) << (31 - bi))
                h = hsum(kc ^ imin)
                return jnp.where(h >= target, kc, k)
            kk = lax.fori_loop(0, 32, bstep, jnp.int32(0))

            ks = kk ^ imin
            fbits = jnp.where(kk < 0, kk & (~imin), ~kk)
            fb_v = jnp.full((_L,), fbits, jnp.int32)
            vstar_v = plsc.bitcast(fb_v, jnp.float32)
            vstar = jnp.max(vstar_v)
            estar_v = jnp.exp(vstar_v - m)

            def ggt_f(si, acc):
                valid = cnt > si
                sk = cand_k[pl.ds(si * _L, _L)]
                ee = cand_e[pl.ds(si * _L, _L)]
                sel = jnp.logical_and(valid, sk > ks)
                return acc + jnp.where(sel, ee, 0.0)
            g_gt = g_ab + jnp.sum(lax.fori_loop(0, maxcnt, ggt_f, zeros))
            x = jnp.max((t - g_gt) / estar_v)
            jt = x.astype(jnp.int32).astype(jnp.float32)
            j = jt + (jt < x).astype(jnp.float32)

            def teq(ic):
                def inner(si, acc):
                    valid = cnt > si
                    sk = cand_k[pl.ds(si * _L, _L)]
                    ii = cand_i[pl.ds(si * _L, _L)]
                    sel = jnp.logical_and(
                        valid, jnp.logical_and(sk == ks, ii <= ic))
                    return acc + jnp.where(sel, 1.0, 0.0)
                return jnp.sum(lax.fori_loop(0, maxcnt, inner, zeros))

            def tstep(bi, iacc):
                ic = iacc | (jnp.int32(1) << (12 - bi))
                return jnp.where(teq(ic) < j, ic, iacc)
            iacc = lax.fori_loop(0, 13, tstep, jnp.int32(0))
            bound = jnp.where(teq(iacc) < j, iacc + 1, iacc)

            def pf(i, c):
                for u in range(4):
                    off = i * (4 * _L) + u * _L
                    zv = zrow[pl.ds(off, _L)]
                    b = jnp.clip(((zv - lo) * scale).astype(jnp.int32),
                                 0, _B1 - 1)
                    plsc.store_scatter(hist, [b * _L + lane], zeros)
                    idxv = (i * 4 + u) * _L + lane
                    keep = jnp.logical_or(
                        zv > vstar,
                        jnp.logical_and(zv == vstar, idxv <= bound))
                    orow[pl.ds(off, _L)] = jnp.where(
                        keep, jnp.maximum(zv, 0.0), 0.0)
                return c
            lax.fori_loop(0, _NV // 4, pf, 0)

            pltpu.sync_copy(orow, out_hbm.at[row])
            return carry

        lax.fori_loop(0, _RPW, do_row, 0)

    return sc_kernel


_sc_cache = []


@jax.jit
def kernel(z):
    if not _sc_cache:
        _sc_cache.append(_make_sc_kernel())
    return _sc_cache[0](z)
